# Initial kernel scaffold; baseline (speedup 1.0000x reference)
import jax
import jax.numpy as jnp
from jax.experimental import pallas as pl


def kernel(x, edge_index, W, b, gamma, beta):
    raise NotImplementedError("write your pallas kernel here")


# speedup vs baseline: 15.3687x; 15.3687x over previous
import dataclasses
import functools

import jax
import jax.numpy as jnp
from jax import lax
from jax.experimental import pallas as pl
from jax.experimental.pallas import tpu as pltpu
from jax.experimental.pallas import tpu_sc as plsc

N = 10000
E = 320000
D = 128
BN_EPS = 1e-5

NC = 2
NS = 16
NW = NC * NS
EPW = E // NW
K = 80
CH = EPW // K
NP = 10240
RPT = NP // NS
DH = D // 2

_mesh = plsc.VectorSubcoreMesh(core_axis_name="c", subcore_axis_name="s")

_sc_params = pltpu.CompilerParams()
if "needs_layout_passes" in pltpu.CompilerParams.__dataclass_fields__:
    _sc_params = dataclasses.replace(_sc_params, needs_layout_passes=False)
_sc_agg_params = dataclasses.replace(_sc_params, use_tc_tiling_on_sc=False)


@functools.partial(
    pl.kernel,
    out_type=jax.ShapeDtypeStruct((NW, 2, N), jnp.float32),
    mesh=_mesh,
    scratch_types=[
        pltpu.VMEM((CH, K), jnp.int32),
        pltpu.VMEM((CH, K), jnp.int32),
        pltpu.VMEM((N,), jnp.float32),
        pltpu.VMEM((N,), jnp.float32),
    ],
    compiler_params=_sc_params,
)
def _sc_degrees(src_hbm, dst_hbm, out_hbm, src_v, dst_v, hs_v, hd_v):
    c = lax.axis_index("c")
    s = lax.axis_index("s")
    wid = s * NC + c
    pltpu.sync_copy(src_hbm.at[wid], src_v)
    pltpu.sync_copy(dst_hbm.at[wid], dst_v)

    zeros = jnp.zeros((16,), jnp.float32)

    @pl.loop(0, N // 16)
    def _(i):
        hs_v[pl.ds(i * 16, 16)] = zeros
        hd_v[pl.ds(i * 16, 16)] = zeros

    ones = jnp.ones((16,), jnp.float32)

    @pl.loop(0, CH)
    def _(i):
        for j in range(K // 16):
            si = src_v[i, pl.ds(j * 16, 16)]
            di = dst_v[i, pl.ds(j * 16, 16)]
            plsc.addupdate_scatter(hs_v, [si], ones)
            plsc.addupdate_scatter(hd_v, [di], ones)

    pltpu.sync_copy(hs_v, out_hbm.at[wid, 0])
    pltpu.sync_copy(hd_v, out_hbm.at[wid, 1])


@functools.partial(
    pl.kernel,
    out_type=jax.ShapeDtypeStruct((NC, 2, NP, DH), jnp.float32),
    mesh=_mesh,
    scratch_types=[
        pltpu.VMEM((CH, K), jnp.int32),
        pltpu.VMEM((CH, K), jnp.int32),
        pltpu.VMEM((K, DH), jnp.float32),
        pltpu.VMEM((K, DH), jnp.float32),
        pltpu.VMEM_SHARED((NP, DH), jnp.float32),
        pltpu.SemaphoreType.DMA,
        pltpu.SemaphoreType.DMA,
    ],
    compiler_params=_sc_agg_params,
)
def _sc_aggregate(xw0_hbm, xw1_hbm, src_hbm, dst_hbm, zero_hbm, out_hbm,
                  src_v, dst_v, rows0, rows1, acc_sh, sem0, sem1):
    c = lax.axis_index("c")
    s = lax.axis_index("s")
    wid = s * NC + c
    pltpu.sync_copy(src_hbm.at[wid], src_v)
    pltpu.sync_copy(dst_hbm.at[wid], dst_v)

    for h, xw_hbm in ((0, xw0_hbm), (1, xw1_hbm)):
        pltpu.sync_copy(zero_hbm.at[pl.ds(s * RPT, RPT)],
                        acc_sh.at[pl.ds(s * RPT, RPT)])
        plsc.subcore_barrier()

        pltpu.async_copy(xw_hbm.at[src_v.at[0]], rows0, sem0)

        @pl.loop(0, (CH - 1) // 2)
        def _(j):
            i = 2 * j
            pltpu.make_async_copy(xw_hbm.at[src_v.at[i]], rows0, sem0).wait()
            pltpu.async_copy(xw_hbm.at[src_v.at[i + 1]], rows1, sem1)
            pltpu.sync_copy(rows0, acc_sh.at[dst_v.at[i]], add=True)
            pltpu.make_async_copy(xw_hbm.at[src_v.at[i + 1]], rows1,
                                  sem1).wait()
            pltpu.async_copy(xw_hbm.at[src_v.at[i + 2]], rows0, sem0)
            pltpu.sync_copy(rows1, acc_sh.at[dst_v.at[i + 1]], add=True)

        pltpu.make_async_copy(xw_hbm.at[src_v.at[CH - 1]], rows0, sem0).wait()
        pltpu.sync_copy(rows0, acc_sh.at[dst_v.at[CH - 1]], add=True)

        plsc.subcore_barrier()
        pltpu.sync_copy(acc_sh.at[pl.ds(s * RPT, RPT)],
                        out_hbm.at[c, h, pl.ds(s * RPT, RPT)])


def _tc_project_body(x_ref, w_ref, h_ref, o0_ref, o1_ref):
    deg = jnp.sum(h_ref[:, 0, :], axis=0)
    norm = lax.rsqrt(jnp.maximum(deg, 1.0))
    xw = jnp.dot(x_ref[...], w_ref[...], preferred_element_type=jnp.float32)
    xws = xw * norm[:, None]
    o0_ref[...] = xws[:, :DH]
    o1_ref[...] = xws[:, DH:]


_tc_project = pl.pallas_call(
    _tc_project_body,
    out_shape=[jax.ShapeDtypeStruct((N, DH), jnp.float32),
               jax.ShapeDtypeStruct((N, DH), jnp.float32)],
)


def _tc_finish_body(parts_ref, h_ref, b_ref, g_ref, bt_ref, o_ref):
    agg0 = parts_ref[0, 0, :N, :] + parts_ref[1, 0, :N, :]
    agg1 = parts_ref[0, 1, :N, :] + parts_ref[1, 1, :N, :]
    agg = jnp.concatenate([agg0, agg1], axis=1)
    deg_in = jnp.sum(h_ref[:, 1, :], axis=0)
    nd = lax.rsqrt(jnp.maximum(deg_in, 1.0))
    hpre = agg * nd[:, None] + b_ref[...]
    mean = jnp.mean(hpre, axis=0, keepdims=True)
    var = jnp.mean((hpre - mean) ** 2, axis=0, keepdims=True)
    o_ref[...] = (hpre - mean) * lax.rsqrt(var + BN_EPS) * g_ref[...] + bt_ref[...]


_tc_finish = pl.pallas_call(
    _tc_finish_body,
    out_shape=jax.ShapeDtypeStruct((N, D), jnp.float32),
)


def kernel(x, edge_index, W, b, gamma, beta):
    src = edge_index[0].reshape(NW, CH, K)
    dst = edge_index[1].reshape(NW, CH, K)
    hists = _sc_degrees(src, dst)
    xw0, xw1 = _tc_project(x, W, hists)
    zeros = jnp.zeros((NP, DH), jnp.float32)
    parts = _sc_aggregate(xw0, xw1, src, dst, zeros)
    return _tc_finish(parts, hists, b.reshape(1, D), gamma.reshape(1, D),
                      beta.reshape(1, D))


# speedup vs baseline: 23.3565x; 1.5197x over previous
import dataclasses
import functools

import jax
import jax.numpy as jnp
from jax import lax
from jax.experimental import pallas as pl
from jax.experimental.pallas import tpu as pltpu
from jax.experimental.pallas import tpu_sc as plsc

N = 10000
E = 320000
D = 128
BN_EPS = 1e-5

NC = 2
NS = 16
NW = NC * NS
EPW = E // NW
K = 80
CH = EPW // K
NP = 10240
RPT = NP // NS
DH = D // 2
KA = 128
EPT = 10240
CHA = EPT // KA
NSLOT = 4

_mesh = plsc.VectorSubcoreMesh(core_axis_name="c", subcore_axis_name="s")

_sc_params = pltpu.CompilerParams()
if "needs_layout_passes" in pltpu.CompilerParams.__dataclass_fields__:
    _sc_params = dataclasses.replace(_sc_params, needs_layout_passes=False)
_sc_agg_params = dataclasses.replace(_sc_params, use_tc_tiling_on_sc=False)


@functools.partial(
    pl.kernel,
    out_type=jax.ShapeDtypeStruct((NW, 2, N), jnp.float32),
    mesh=_mesh,
    scratch_types=[
        pltpu.VMEM((CH, K), jnp.int32),
        pltpu.VMEM((CH, K), jnp.int32),
        pltpu.VMEM((N,), jnp.float32),
        pltpu.VMEM((N,), jnp.float32),
    ],
    compiler_params=_sc_params,
)
def _sc_degrees(src_hbm, dst_hbm, out_hbm, src_v, dst_v, hs_v, hd_v):
    c = lax.axis_index("c")
    s = lax.axis_index("s")
    wid = s * NC + c
    pltpu.sync_copy(src_hbm.at[wid], src_v)
    pltpu.sync_copy(dst_hbm.at[wid], dst_v)

    zeros = jnp.zeros((16,), jnp.float32)

    @pl.loop(0, N // 16)
    def _(i):
        hs_v[pl.ds(i * 16, 16)] = zeros
        hd_v[pl.ds(i * 16, 16)] = zeros

    ones = jnp.ones((16,), jnp.float32)

    @pl.loop(0, CH)
    def _(i):
        for j in range(K // 16):
            si = src_v[i, pl.ds(j * 16, 16)]
            di = dst_v[i, pl.ds(j * 16, 16)]
            plsc.addupdate_scatter(hs_v, [si], ones)
            plsc.addupdate_scatter(hd_v, [di], ones)

    pltpu.sync_copy(hs_v, out_hbm.at[wid, 0])
    pltpu.sync_copy(hd_v, out_hbm.at[wid, 1])


@functools.partial(
    pl.kernel,
    out_type=jax.ShapeDtypeStruct((NC, 2, NP, DH), jnp.float32),
    mesh=_mesh,
    scratch_types=[
        pltpu.VMEM((CHA, KA), jnp.int32),
        pltpu.VMEM((CHA, KA), jnp.int32),
        [pltpu.VMEM((KA, DH), jnp.float32) for _ in range(NSLOT)],
        pltpu.VMEM((128, DH), jnp.float32),
        pltpu.VMEM_SHARED((NP, DH), jnp.float32),
        [pltpu.SemaphoreType.DMA for _ in range(NSLOT)],
        [pltpu.SemaphoreType.DMA for _ in range(NSLOT)],
    ],
    compiler_params=_sc_agg_params,
)
def _sc_aggregate(xw0_hbm, xw1_hbm, src_hbm, dst_hbm, out_hbm,
                  src_v, dst_v, bufs, zb, acc_sh, gsems, ssems):
    c = lax.axis_index("c")
    s = lax.axis_index("s")
    wid = s * NC + c
    pltpu.sync_copy(src_hbm.at[wid], src_v)
    pltpu.sync_copy(dst_hbm.at[wid], dst_v)

    zvec = jnp.zeros((16,), jnp.float32)

    @pl.loop(0, 128)
    def _(r):
        for q in range(DH // 16):
            zb[r, pl.ds(q * 16, 16)] = zvec

    for h, xw_hbm in ((0, xw0_hbm), (1, xw1_hbm)):
        for k in range(RPT // 128):
            pltpu.sync_copy(zb, acc_sh.at[pl.ds(s * RPT + k * 128, 128)])
        plsc.subcore_barrier()

        for t in range(NSLOT):
            pltpu.async_copy(xw_hbm.at[src_v.at[t]], bufs[t], gsems[t])

        @pl.loop(0, CHA // NSLOT - 1)
        def _(j):
            i = NSLOT * j
            sc_descs = []
            for t in range(NSLOT):
                pltpu.make_async_copy(xw_hbm.at[src_v.at[i + t]], bufs[t],
                                      gsems[t]).wait()
                sc_descs.append(pltpu.async_copy(
                    bufs[t], acc_sh.at[dst_v.at[i + t]], ssems[t], add=True))
            for t in range(NSLOT):
                sc_descs[t].wait()
                pltpu.async_copy(xw_hbm.at[src_v.at[i + NSLOT + t]], bufs[t],
                                 gsems[t])

        i0 = CHA - NSLOT
        sc_descs = []
        for t in range(NSLOT):
            pltpu.make_async_copy(xw_hbm.at[src_v.at[i0 + t]], bufs[t],
                                  gsems[t]).wait()
            sc_descs.append(pltpu.async_copy(
                bufs[t], acc_sh.at[dst_v.at[i0 + t]], ssems[t], add=True))
        for t in range(NSLOT):
            sc_descs[t].wait()

        plsc.subcore_barrier()
        pltpu.sync_copy(acc_sh.at[pl.ds(s * RPT, RPT)],
                        out_hbm.at[c, h, pl.ds(s * RPT, RPT)])
        plsc.subcore_barrier()


def _tc_project_body(x_ref, w_ref, h_ref, o0_ref, o1_ref):
    deg = jnp.sum(h_ref[:, 0, :], axis=0)
    norm = lax.rsqrt(jnp.maximum(deg, 1.0))
    xw = jnp.dot(x_ref[...], w_ref[...], preferred_element_type=jnp.float32)
    xws = xw * norm[:, None]
    o0_ref[...] = xws[:, :DH]
    o1_ref[...] = xws[:, DH:]


_tc_project = pl.pallas_call(
    _tc_project_body,
    out_shape=[jax.ShapeDtypeStruct((N, DH), jnp.float32),
               jax.ShapeDtypeStruct((N, DH), jnp.float32)],
)


def _tc_finish_body(parts_ref, h_ref, b_ref, g_ref, bt_ref, o_ref):
    agg0 = parts_ref[0, 0, :N, :] + parts_ref[1, 0, :N, :]
    agg1 = parts_ref[0, 1, :N, :] + parts_ref[1, 1, :N, :]
    agg = jnp.concatenate([agg0, agg1], axis=1)
    deg_in = jnp.sum(h_ref[:, 1, :], axis=0)
    nd = lax.rsqrt(jnp.maximum(deg_in, 1.0))
    hpre = agg * nd[:, None] + b_ref[...]
    mean = jnp.mean(hpre, axis=0, keepdims=True)
    var = jnp.mean((hpre - mean) ** 2, axis=0, keepdims=True)
    o_ref[...] = (hpre - mean) * lax.rsqrt(var + BN_EPS) * g_ref[...] + bt_ref[...]


_tc_finish = pl.pallas_call(
    _tc_finish_body,
    out_shape=jax.ShapeDtypeStruct((N, D), jnp.float32),
)


def kernel(x, edge_index, W, b, gamma, beta):
    src = edge_index[0].reshape(NW, CH, K)
    dst = edge_index[1].reshape(NW, CH, K)
    hists = _sc_degrees(src, dst)
    xw0, xw1 = _tc_project(x, W, hists)
    npad = EPT - EPW
    pad_src = jnp.broadcast_to(jnp.arange(npad, dtype=jnp.int32) % N,
                               (NW, npad))
    pad_dst = jnp.broadcast_to(N + jnp.arange(npad, dtype=jnp.int32) % (NP - N),
                               (NW, npad))
    src_p = jnp.concatenate([edge_index[0].reshape(NW, EPW), pad_src],
                            axis=1).reshape(NW, CHA, KA)
    dst_p = jnp.concatenate([edge_index[1].reshape(NW, EPW), pad_dst],
                            axis=1).reshape(NW, CHA, KA)
    parts = _sc_aggregate(xw0, xw1, src_p, dst_p)
    return _tc_finish(parts, hists, b.reshape(1, D), gamma.reshape(1, D),
                      beta.reshape(1, D))
